# Initial kernel scaffold; baseline (speedup 1.0000x reference)
import jax
import jax.numpy as jnp
from jax.experimental import pallas as pl


def kernel(x, edge_index, d_indices, d_values, weight, filter1, bias, W2, a_src2, a_dst2, b2, W3, a_src3, a_dst3, b3):
    raise NotImplementedError("write your pallas kernel here")


# speedup vs baseline: 4.3299x; 4.3299x over previous
import functools

import jax
import jax.numpy as jnp
from jax import lax
from jax.experimental import pallas as pl
from jax.experimental.pallas import tpu as pltpu
from jax.experimental.pallas import tpu_sc as plsc

N = 10000
F = 128
E = 320000
NNZ = 320000

NC = 2
NS = 16
NW = NC * NS
L = 16

NPAD = 10240
RPT = NPAD // NS
C = 80

EPT = NNZ // NW
NCH = EPT // C

EG = E + N
EGP = 330240
EGPT = EGP // NW
NCHG = EGPT // C
EROWS = EGP // 128

_f32 = jnp.float32


@functools.lru_cache(maxsize=1)
def _sc_mesh():
    return plsc.VectorSubcoreMesh(core_axis_name="c", subcore_axis_name="s")


def _wid_cid_sid():
    cid = lax.axis_index("c")
    sid = lax.axis_index("s")
    return sid * NC + cid, cid, sid


def _zero_acc2d(zeros_h, acc, sid):
    for z in range(RPT // 128):
        pltpu.sync_copy(zeros_h, acc.at[pl.ds(sid * RPT + z * 128, 128)])


def _zero_acc1d(zvec_h, acc, sid):
    for z in range(RPT // 128):
        pltpu.sync_copy(zvec_h, acc.at[pl.ds(sid * RPT + z * 128, 128)])


def _scale_rows(rows, coef, n_rows):
    def body(r, _):
        bc = plsc.load_gather(coef, [jnp.full((L,), r, jnp.int32)])
        for j in range(F // L):
            sl = pl.ds(j * L, L)
            rows[r, sl] = rows[r, sl] * bc
        return 0

    lax.fori_loop(0, n_rows, body, 0)


def _spmm3_body(r1_h, c1_h, v1_h, r2_h, c2_h, v2_h, r3_h, c3_h, v3_h,
                xw_h, zeros_h,
                y1a, y1b, y2a, y2b, y3a, y3b,
                ridx, cidx, vv, rows, acc, sem):
    wid, cid, sid = _wid_cid_sid()
    outs = [(y1a, y1b), (y2a, y2b), (y3a, y3b)]
    coo = [(r1_h, c1_h, v1_h), (r2_h, c2_h, v2_h), (r3_h, c3_h, v3_h)]
    for k in range(3):
        r_h, c_h, v_h = coo[k]
        _zero_acc2d(zeros_h, acc, sid)
        plsc.subcore_barrier()

        def chunk(ch, _):
            base = wid * EPT + ch * C
            pltpu.sync_copy(r_h.at[pl.ds(base, C)], ridx)
            pltpu.sync_copy(c_h.at[pl.ds(base, C)], cidx)
            pltpu.sync_copy(v_h.at[pl.ds(base, C)], vv)
            pltpu.async_copy(xw_h.at[cidx], rows, sem).wait()
            _scale_rows(rows, vv, C)
            pltpu.sync_copy(rows, acc.at[ridx], add=True)
            return 0

        lax.fori_loop(0, NCH, chunk, 0)
        plsc.subcore_barrier()
        oa, ob = outs[k]
        sl = pl.ds(sid * RPT, RPT)
        @pl.when(cid == 0)
        def _():
            pltpu.sync_copy(acc.at[sl], oa.at[sl])
        @pl.when(cid == 1)
        def _():
            pltpu.sync_copy(acc.at[sl], ob.at[sl])
        plsc.subcore_barrier()


@functools.lru_cache(maxsize=1)
def _spmm3():
  return functools.partial(
    pl.kernel,
    out_type=[jax.ShapeDtypeStruct((NPAD, F), _f32) for _ in range(6)],
    mesh=_sc_mesh(),
    compiler_params=pltpu.CompilerParams(needs_layout_passes=False),
    scratch_types=[
        pltpu.VMEM((C,), jnp.int32),
        pltpu.VMEM((C,), jnp.int32),
        pltpu.VMEM((C,), _f32),
        pltpu.VMEM((C, F), _f32),
        pltpu.VMEM_SHARED((NPAD, F), _f32),
        pltpu.SemaphoreType.DMA,
    ],
)(_spmm3_body)


def _alpha_body(srcg_h, dstg_h, ss2_h, sd2_h, ss3_h, sd3_h,
                a2_h, a3_h,
                sidx, didx, bss2, bsd2, bss3, bsd3, eb2, eb3, sem):
    wid, _, _ = _wid_cid_sid()

    def chunk(ch, _):
        base = wid * EGPT + ch * C
        pltpu.sync_copy(srcg_h.at[pl.ds(base, C)], sidx)
        pltpu.sync_copy(dstg_h.at[pl.ds(base, C)], didx)
        pltpu.async_copy(ss2_h.at[sidx], bss2, sem).wait()
        pltpu.async_copy(sd2_h.at[didx], bsd2, sem).wait()
        pltpu.async_copy(ss3_h.at[sidx], bss3, sem).wait()
        pltpu.async_copy(sd3_h.at[didx], bsd3, sem).wait()
        for t in range(C // L):
            sl = pl.ds(t * L, L)
            z2 = bss2[sl] + bsd2[sl]
            eb2[sl] = jnp.maximum(z2, 0.2 * z2)
            z3 = bss3[sl] + bsd3[sl]
            eb3[sl] = jnp.maximum(z3, 0.2 * z3)
        pltpu.sync_copy(eb2, a2_h.at[pl.ds(base, C)])
        pltpu.sync_copy(eb3, a3_h.at[pl.ds(base, C)])
        return 0

    lax.fori_loop(0, NCHG, chunk, 0)


@functools.lru_cache(maxsize=1)
def _alpha():
  return functools.partial(
    pl.kernel,
    out_type=[jax.ShapeDtypeStruct((EGP,), _f32) for _ in range(2)],
    mesh=_sc_mesh(),
    compiler_params=pltpu.CompilerParams(needs_layout_passes=False),
    scratch_types=[
        pltpu.VMEM((C,), jnp.int32),
        pltpu.VMEM((C,), jnp.int32),
        pltpu.VMEM((C,), _f32),
        pltpu.VMEM((C,), _f32),
        pltpu.VMEM((C,), _f32),
        pltpu.VMEM((C,), _f32),
        pltpu.VMEM((C,), _f32),
        pltpu.VMEM((C,), _f32),
        pltpu.SemaphoreType.DMA,
    ],
)(_alpha_body)


def _gagg_body(srcg_h, dstg_h, e2_h, e3_h, h2_h, h3_h, zeros_h, zvec_h,
               yg2a, yg2b, yg3a, yg3b, dn2a, dn2b, dn3a, dn3b,
               sidx, didx, ev, rows, acc, dn, sem):
    wid, cid, sid = _wid_cid_sid()
    phases = [(e2_h, h2_h, yg2a, yg2b, dn2a, dn2b),
              (e3_h, h3_h, yg3a, yg3b, dn3a, dn3b)]
    for e_h, h_h, oa, ob, da, db in phases:
        _zero_acc2d(zeros_h, acc, sid)
        _zero_acc1d(zvec_h, dn, sid)
        plsc.subcore_barrier()

        def chunk(ch, _):
            base = wid * EGPT + ch * C
            pltpu.sync_copy(srcg_h.at[pl.ds(base, C)], sidx)
            pltpu.sync_copy(dstg_h.at[pl.ds(base, C)], didx)
            pltpu.sync_copy(e_h.at[pl.ds(base, C)], ev)
            pltpu.sync_copy(ev, dn.at[didx], add=True)
            pltpu.async_copy(h_h.at[sidx], rows, sem).wait()
            _scale_rows(rows, ev, C)
            pltpu.sync_copy(rows, acc.at[didx], add=True)
            return 0

        lax.fori_loop(0, NCHG, chunk, 0)
        plsc.subcore_barrier()
        sl = pl.ds(sid * RPT, RPT)
        @pl.when(cid == 0)
        def _():
            pltpu.sync_copy(acc.at[sl], oa.at[sl])
            pltpu.sync_copy(dn.at[sl], da.at[sl])
        @pl.when(cid == 1)
        def _():
            pltpu.sync_copy(acc.at[sl], ob.at[sl])
            pltpu.sync_copy(dn.at[sl], db.at[sl])
        plsc.subcore_barrier()


@functools.lru_cache(maxsize=1)
def _gagg():
  return functools.partial(
    pl.kernel,
    out_type=[jax.ShapeDtypeStruct((NPAD, F), _f32) for _ in range(4)]
             + [jax.ShapeDtypeStruct((NPAD,), _f32) for _ in range(4)],
    mesh=_sc_mesh(),
    compiler_params=pltpu.CompilerParams(needs_layout_passes=False),
    scratch_types=[
        pltpu.VMEM((C,), jnp.int32),
        pltpu.VMEM((C,), jnp.int32),
        pltpu.VMEM((C,), _f32),
        pltpu.VMEM((C, F), _f32),
        pltpu.VMEM_SHARED((NPAD, F), _f32),
        pltpu.VMEM_SHARED((NPAD,), _f32),
        pltpu.SemaphoreType.DMA,
    ],
)(_gagg_body)


def _spmmt_body(r1_h, c1_h, v1_h, r2_h, c2_h, v2_h, r3_h, c3_h, v3_h,
                x1_h, x2_h, x3_h, zeros_h,
                opa, opb,
                ridx, cidx, vv, rows, acc, sem):
    wid, cid, sid = _wid_cid_sid()
    _zero_acc2d(zeros_h, acc, sid)
    plsc.subcore_barrier()
    coo = [(r1_h, c1_h, v1_h), (r2_h, c2_h, v2_h), (r3_h, c3_h, v3_h)]
    for k, xk_h in enumerate([x1_h, x2_h, x3_h]):
        r_h, c_h, v_h = coo[k]

        def chunk(ch, _):
            base = wid * EPT + ch * C
            pltpu.sync_copy(r_h.at[pl.ds(base, C)], ridx)
            pltpu.sync_copy(c_h.at[pl.ds(base, C)], cidx)
            pltpu.sync_copy(v_h.at[pl.ds(base, C)], vv)
            pltpu.async_copy(xk_h.at[ridx], rows, sem).wait()
            _scale_rows(rows, vv, C)
            pltpu.sync_copy(rows, acc.at[cidx], add=True)
            return 0

        lax.fori_loop(0, NCH, chunk, 0)
    plsc.subcore_barrier()
    sl = pl.ds(sid * RPT, RPT)
    @pl.when(cid == 0)
    def _():
        pltpu.sync_copy(acc.at[sl], opa.at[sl])
    @pl.when(cid == 1)
    def _():
        pltpu.sync_copy(acc.at[sl], opb.at[sl])


@functools.lru_cache(maxsize=1)
def _spmmt():
  return functools.partial(
    pl.kernel,
    out_type=[jax.ShapeDtypeStruct((NPAD, F), _f32) for _ in range(2)],
    mesh=_sc_mesh(),
    compiler_params=pltpu.CompilerParams(needs_layout_passes=False),
    scratch_types=[
        pltpu.VMEM((C,), jnp.int32),
        pltpu.VMEM((C,), jnp.int32),
        pltpu.VMEM((C,), _f32),
        pltpu.VMEM((C, F), _f32),
        pltpu.VMEM_SHARED((NPAD, F), _f32),
        pltpu.SemaphoreType.DMA,
    ],
)(_spmmt_body)


_BLK = 1024
_GRID = NPAD // _BLK


def _mm_body(x_r, w_r, o_r):
    o_r[...] = jnp.dot(x_r[...], w_r[...], preferred_element_type=_f32)


def _tc_matmul(xp, w):
    return pl.pallas_call(
        _mm_body,
        grid=(_GRID,),
        in_specs=[pl.BlockSpec((_BLK, F), lambda i: (i, 0)),
                  pl.BlockSpec((F, F), lambda i: (0, 0))],
        out_specs=pl.BlockSpec((_BLK, F), lambda i: (i, 0)),
        out_shape=jax.ShapeDtypeStruct((NPAD, F), _f32),
    )(xp, w)


def _mid_body(y1a_r, y1b_r, y2a_r, y2b_r, y3a_r, y3b_r, filt_r, w2_r, w3_r,
              x1_r, h2_r, h3_r):
    x1_r[...] = filt_r[...] * (y1a_r[...] + y1b_r[...])
    h2_r[...] = jnp.dot(y2a_r[...] + y2b_r[...], w2_r[...],
                        preferred_element_type=_f32)
    h3_r[...] = jnp.dot(y3a_r[...] + y3b_r[...], w3_r[...],
                        preferred_element_type=_f32)


def _tc_mid(y1a, y1b, y2a, y2b, y3a, y3b, filt, w2, w3):
    bs = pl.BlockSpec((_BLK, F), lambda i: (i, 0))
    ws = pl.BlockSpec((F, F), lambda i: (0, 0))
    return pl.pallas_call(
        _mid_body,
        grid=(_GRID,),
        in_specs=[bs, bs, bs, bs, bs, bs,
                  pl.BlockSpec((_BLK, 1), lambda i: (i, 0)), ws, ws],
        out_specs=[bs, bs, bs],
        out_shape=[jax.ShapeDtypeStruct((NPAD, F), _f32) for _ in range(3)],
    )(y1a, y1b, y2a, y2b, y3a, y3b, filt, w2, w3)


def _tab_body(h2_r, h3_r, as2_r, ad2_r, as3_r, ad3_r,
              ss2_r, sd2_r, ss3_r, sd3_r, m_r):
    h2 = h2_r[...]
    h3 = h3_r[...]
    ss2 = jnp.sum(h2 * as2_r[...][None, :], axis=1)
    sd2 = jnp.sum(h2 * ad2_r[...][None, :], axis=1)
    ss3 = jnp.sum(h3 * as3_r[...][None, :], axis=1)
    sd3 = jnp.sum(h3 * ad3_r[...][None, :], axis=1)
    ss2_r[...] = ss2
    sd2_r[...] = sd2
    ss3_r[...] = ss3
    sd3_r[...] = sd3
    m2 = jnp.max(ss2) + jnp.max(sd2)
    m3 = jnp.max(ss3) + jnp.max(sd3)
    lane = lax.broadcasted_iota(jnp.int32, (F,), 0)
    m_r[...] = jnp.where(lane == 0, m2, jnp.where(lane == 1, m3, 0.0))


def _tc_tables(h2, h3, as2, ad2, as3, ad3):
    full = pl.BlockSpec((NPAD, F), lambda: (0, 0))
    vec = pl.BlockSpec((F,), lambda: (0,))
    tab = pl.BlockSpec((NPAD,), lambda: (0,))
    return pl.pallas_call(
        _tab_body,
        in_specs=[full, full, vec, vec, vec, vec],
        out_specs=[tab, tab, tab, tab, pl.BlockSpec((F,), lambda: (0,))],
        out_shape=[jax.ShapeDtypeStruct((NPAD,), _f32) for _ in range(4)]
                  + [jax.ShapeDtypeStruct((F,), _f32)],
    )(h2, h3, as2, ad2, as3, ad3)


def _exp_body(a2_r, a3_r, m_r, e2_r, e3_r):
    m = m_r[...]
    e2_r[...] = jnp.exp(a2_r[...] - m[0])
    e3_r[...] = jnp.exp(a3_r[...] - m[1])


def _tc_exp(a2, a3, m):
    full = pl.BlockSpec((EROWS, 128), lambda: (0, 0))
    return pl.pallas_call(
        _exp_body,
        in_specs=[full, full, pl.BlockSpec((F,), lambda: (0,))],
        out_specs=[full, full],
        out_shape=[jax.ShapeDtypeStruct((EROWS, 128), _f32) for _ in range(2)],
    )(a2, a3, m)


def _norm_body(ya_r, yb_r, da_r, db_r, bias_r, o_r):
    dn = da_r[...] + db_r[...] + 1e-16
    v = (ya_r[...] + yb_r[...]) / dn[:, None] + bias_r[...][None, :]
    o_r[...] = jnp.where(v > 0, v, jnp.exp(jnp.minimum(v, 0.0)) - 1.0)


def _tc_norm_elu(ya, yb, da, db, bias):
    bs = pl.BlockSpec((_BLK, F), lambda i: (i, 0))
    ds1 = pl.BlockSpec((_BLK,), lambda i: (i,))
    return pl.pallas_call(
        _norm_body,
        grid=(_GRID,),
        in_specs=[bs, bs, ds1, ds1, pl.BlockSpec((F,), lambda i: (0,))],
        out_specs=bs,
        out_shape=jax.ShapeDtypeStruct((NPAD, F), _f32),
    )(ya, yb, da, db, bias)


def _fin_body(a_r, b_r, bias_r, o_r):
    o_r[...] = a_r[...] + b_r[...] + bias_r[...][None, :]


def _tc_final(a, b, bias):
    bs = pl.BlockSpec((1000, F), lambda i: (i, 0))
    return pl.pallas_call(
        _fin_body,
        grid=(10,),
        in_specs=[bs, bs, pl.BlockSpec((F,), lambda i: (0,))],
        out_specs=bs,
        out_shape=jax.ShapeDtypeStruct((N, F), _f32),
    )(a, b, bias)


def kernel(x, edge_index, d_indices, d_values, weight, filter1, bias,
           W2, a_src2, a_dst2, b2, W3, a_src3, a_dst3, b3):
    n = x.shape[0]
    zeros = jnp.zeros((128, F), _f32)
    zvec = jnp.zeros((128,), _f32)
    dr = [d_indices[k, 0] for k in range(3)]
    dc = [d_indices[k, 1] for k in range(3)]
    dv = [d_values[k] for k in range(3)]
    coo = [a for k in range(3) for a in (dr[k], dc[k], dv[k])]
    xp = jnp.zeros((NPAD, F), _f32).at[:n].set(x)
    filt = jnp.zeros((NPAD, 1), _f32).at[:n].set(filter1)
    loop = jnp.arange(n, dtype=jnp.int32)
    padlen = EGP - EG
    srcg = jnp.concatenate([edge_index[0], loop,
                            jnp.zeros((padlen,), jnp.int32)])
    dstg = jnp.concatenate([edge_index[1], loop,
                            jnp.full((padlen,), n, jnp.int32)])

    xw = _tc_matmul(xp, weight)
    y1a, y1b, y2a, y2b, y3a, y3b = _spmm3()(*coo, xw, zeros)
    x1, h2, h3 = _tc_mid(y1a, y1b, y2a, y2b, y3a, y3b, filt, W2, W3)
    ss2, sd2, ss3, sd3, m = _tc_tables(h2, h3, a_src2, a_dst2, a_src3, a_dst3)
    a2, a3 = _alpha()(srcg, dstg, ss2, sd2, ss3, sd3)
    e2m, e3m = _tc_exp(a2.reshape(EROWS, 128), a3.reshape(EROWS, 128), m)
    e2 = e2m.reshape(EGP)
    e3 = e3m.reshape(EGP)
    (yg2a, yg2b, yg3a, yg3b,
     dn2a, dn2b, dn3a, dn3b) = _gagg()(srcg, dstg, e2, e3, h2, h3,
                                       zeros, zvec)
    x2f = _tc_norm_elu(yg2a, yg2b, dn2a, dn2b, b2)
    x3f = _tc_norm_elu(yg3a, yg3b, dn3a, dn3b, b3)
    opa, opb = _spmmt()(*coo, x1, x2f, x3f, zeros)
    return _tc_final(opa, opb, bias)


# speedup vs baseline: 6.0470x; 1.3966x over previous
import functools

import jax
import jax.numpy as jnp
from jax import lax
from jax.experimental import pallas as pl
from jax.experimental.pallas import tpu as pltpu
from jax.experimental.pallas import tpu_sc as plsc

N = 10000
F = 128
E = 320000
NNZ = 320000

NC = 2
NS = 16
NW = NC * NS
L = 16

NPAD = 10240
RPT = NPAD // NS
C = 80

EPT = NNZ // NW
NCH = EPT // C

EG = E + N
EGP = 330240
EGPT = EGP // NW
NCHG = EGPT // C
EROWS = EGP // 128

_f32 = jnp.float32


@functools.lru_cache(maxsize=1)
def _sc_mesh():
    return plsc.VectorSubcoreMesh(core_axis_name="c", subcore_axis_name="s")


def _wid_cid_sid():
    cid = lax.axis_index("c")
    sid = lax.axis_index("s")
    return sid * NC + cid, cid, sid


def _zero_acc2d(zeros_h, acc, sid):
    for z in range(RPT // 128):
        pltpu.sync_copy(zeros_h, acc.at[pl.ds(sid * RPT + z * 128, 128)])


def _zero_acc1d(zvec_h, acc, sid):
    for z in range(RPT // 128):
        pltpu.sync_copy(zvec_h, acc.at[pl.ds(sid * RPT + z * 128, 128)])


def _scale_rows(rows, coef, n_rows):
    def body(r, _):
        bc = plsc.load_gather(coef, [jnp.full((L,), r, jnp.int32)])
        for j in range(F // L):
            sl = pl.ds(j * L, L)
            rows[r, sl] = rows[r, sl] * bc
        return 0

    lax.fori_loop(0, n_rows, body, 0)


def _spmm3_body(r1_h, c1_h, v1_h, r2_h, c2_h, v2_h, r3_h, c3_h, v3_h,
                xw_h, zeros_h,
                y1a, y1b, y2a, y2b, y3a, y3b,
                ridx, cidx, vv, rows, ridx1, cidx1, vv1, rows1,
                acc, sem, sem1):
    wid, cid, sid = _wid_cid_sid()
    outs = [(y1a, y1b), (y2a, y2b), (y3a, y3b)]
    coo = [(r1_h, c1_h, v1_h), (r2_h, c2_h, v2_h), (r3_h, c3_h, v3_h)]
    for k in range(3):
        r_h, c_h, v_h = coo[k]
        _zero_acc2d(zeros_h, acc, sid)
        plsc.subcore_barrier()

        def chunkpair(g, _):
            b0 = wid * EPT + (2 * g) * C
            b1 = b0 + C
            pltpu.sync_copy(r_h.at[pl.ds(b0, C)], ridx)
            pltpu.sync_copy(c_h.at[pl.ds(b0, C)], cidx)
            pltpu.sync_copy(v_h.at[pl.ds(b0, C)], vv)
            cp0 = pltpu.async_copy(xw_h.at[cidx], rows, sem)
            pltpu.sync_copy(r_h.at[pl.ds(b1, C)], ridx1)
            pltpu.sync_copy(c_h.at[pl.ds(b1, C)], cidx1)
            pltpu.sync_copy(v_h.at[pl.ds(b1, C)], vv1)
            cp1 = pltpu.async_copy(xw_h.at[cidx1], rows1, sem1)
            cp0.wait()
            _scale_rows(rows, vv, C)
            pltpu.sync_copy(rows, acc.at[ridx], add=True)
            cp1.wait()
            _scale_rows(rows1, vv1, C)
            pltpu.sync_copy(rows1, acc.at[ridx1], add=True)
            return 0

        lax.fori_loop(0, NCH // 2, chunkpair, 0)
        base = wid * EPT + (NCH - 1) * C
        pltpu.sync_copy(r_h.at[pl.ds(base, C)], ridx)
        pltpu.sync_copy(c_h.at[pl.ds(base, C)], cidx)
        pltpu.sync_copy(v_h.at[pl.ds(base, C)], vv)
        pltpu.async_copy(xw_h.at[cidx], rows, sem).wait()
        _scale_rows(rows, vv, C)
        pltpu.sync_copy(rows, acc.at[ridx], add=True)
        plsc.subcore_barrier()
        oa, ob = outs[k]
        sl = pl.ds(sid * RPT, RPT)
        @pl.when(cid == 0)
        def _():
            pltpu.sync_copy(acc.at[sl], oa.at[sl])
        @pl.when(cid == 1)
        def _():
            pltpu.sync_copy(acc.at[sl], ob.at[sl])
        plsc.subcore_barrier()


@functools.lru_cache(maxsize=1)
def _spmm3():
  return functools.partial(
    pl.kernel,
    out_type=[jax.ShapeDtypeStruct((NPAD, F), _f32) for _ in range(6)],
    mesh=_sc_mesh(),
    compiler_params=pltpu.CompilerParams(needs_layout_passes=False),
    scratch_types=[
        pltpu.VMEM((C,), jnp.int32),
        pltpu.VMEM((C,), jnp.int32),
        pltpu.VMEM((C,), _f32),
        pltpu.VMEM((C, F), _f32),
        pltpu.VMEM((C,), jnp.int32),
        pltpu.VMEM((C,), jnp.int32),
        pltpu.VMEM((C,), _f32),
        pltpu.VMEM((C, F), _f32),
        pltpu.VMEM_SHARED((NPAD, F), _f32),
        pltpu.SemaphoreType.DMA,
        pltpu.SemaphoreType.DMA,
    ],
)(_spmm3_body)


def _alpha_body(srcg_h, dstg_h, ss2_h, sd2_h, ss3_h, sd3_h,
                a2_h, a3_h,
                sidx, didx, bss2, bsd2, bss3, bsd3, eb2, eb3,
                sem, semb, semc, semd):
    wid, _, _ = _wid_cid_sid()

    def chunk(ch, _):
        base = wid * EGPT + ch * C
        pltpu.sync_copy(srcg_h.at[pl.ds(base, C)], sidx)
        pltpu.sync_copy(dstg_h.at[pl.ds(base, C)], didx)
        c1 = pltpu.async_copy(ss2_h.at[sidx], bss2, sem)
        c2 = pltpu.async_copy(sd2_h.at[didx], bsd2, semb)
        c3 = pltpu.async_copy(ss3_h.at[sidx], bss3, semc)
        c4 = pltpu.async_copy(sd3_h.at[didx], bsd3, semd)
        c1.wait(); c2.wait(); c3.wait(); c4.wait()
        for t in range(C // L):
            sl = pl.ds(t * L, L)
            z2 = bss2[sl] + bsd2[sl]
            eb2[sl] = jnp.maximum(z2, 0.2 * z2)
            z3 = bss3[sl] + bsd3[sl]
            eb3[sl] = jnp.maximum(z3, 0.2 * z3)
        pltpu.sync_copy(eb2, a2_h.at[pl.ds(base, C)])
        pltpu.sync_copy(eb3, a3_h.at[pl.ds(base, C)])
        return 0

    lax.fori_loop(0, NCHG, chunk, 0)


@functools.lru_cache(maxsize=1)
def _alpha():
  return functools.partial(
    pl.kernel,
    out_type=[jax.ShapeDtypeStruct((EGP,), _f32) for _ in range(2)],
    mesh=_sc_mesh(),
    compiler_params=pltpu.CompilerParams(needs_layout_passes=False),
    scratch_types=[
        pltpu.VMEM((C,), jnp.int32),
        pltpu.VMEM((C,), jnp.int32),
        pltpu.VMEM((C,), _f32),
        pltpu.VMEM((C,), _f32),
        pltpu.VMEM((C,), _f32),
        pltpu.VMEM((C,), _f32),
        pltpu.VMEM((C,), _f32),
        pltpu.VMEM((C,), _f32),
        pltpu.SemaphoreType.DMA,
        pltpu.SemaphoreType.DMA,
        pltpu.SemaphoreType.DMA,
        pltpu.SemaphoreType.DMA,
    ],
)(_alpha_body)


def _gagg_body(srcg_h, dstg_h, e2_h, e3_h, h2_h, h3_h, zeros_h, zvec_h,
               yg2a, yg2b, yg3a, yg3b, dn2a, dn2b, dn3a, dn3b,
               sidx, didx, ev, rows, sidx1, didx1, ev1, rows1,
               acc, dn, sem, sem1):
    wid, cid, sid = _wid_cid_sid()
    phases = [(e2_h, h2_h, yg2a, yg2b, dn2a, dn2b),
              (e3_h, h3_h, yg3a, yg3b, dn3a, dn3b)]
    for e_h, h_h, oa, ob, da, db in phases:
        _zero_acc2d(zeros_h, acc, sid)
        _zero_acc1d(zvec_h, dn, sid)
        plsc.subcore_barrier()

        def chunkpair(g, _):
            b0 = wid * EGPT + (2 * g) * C
            b1 = b0 + C
            pltpu.sync_copy(srcg_h.at[pl.ds(b0, C)], sidx)
            pltpu.sync_copy(dstg_h.at[pl.ds(b0, C)], didx)
            pltpu.sync_copy(e_h.at[pl.ds(b0, C)], ev)
            cp0 = pltpu.async_copy(h_h.at[sidx], rows, sem)
            pltpu.sync_copy(srcg_h.at[pl.ds(b1, C)], sidx1)
            pltpu.sync_copy(dstg_h.at[pl.ds(b1, C)], didx1)
            pltpu.sync_copy(e_h.at[pl.ds(b1, C)], ev1)
            cp1 = pltpu.async_copy(h_h.at[sidx1], rows1, sem1)
            pltpu.sync_copy(ev, dn.at[didx], add=True)
            cp0.wait()
            _scale_rows(rows, ev, C)
            pltpu.sync_copy(rows, acc.at[didx], add=True)
            pltpu.sync_copy(ev1, dn.at[didx1], add=True)
            cp1.wait()
            _scale_rows(rows1, ev1, C)
            pltpu.sync_copy(rows1, acc.at[didx1], add=True)
            return 0

        lax.fori_loop(0, NCHG // 2, chunkpair, 0)
        base = wid * EGPT + (NCHG - 1) * C
        pltpu.sync_copy(srcg_h.at[pl.ds(base, C)], sidx)
        pltpu.sync_copy(dstg_h.at[pl.ds(base, C)], didx)
        pltpu.sync_copy(e_h.at[pl.ds(base, C)], ev)
        pltpu.sync_copy(ev, dn.at[didx], add=True)
        pltpu.async_copy(h_h.at[sidx], rows, sem).wait()
        _scale_rows(rows, ev, C)
        pltpu.sync_copy(rows, acc.at[didx], add=True)
        plsc.subcore_barrier()
        sl = pl.ds(sid * RPT, RPT)
        @pl.when(cid == 0)
        def _():
            pltpu.sync_copy(acc.at[sl], oa.at[sl])
            pltpu.sync_copy(dn.at[sl], da.at[sl])
        @pl.when(cid == 1)
        def _():
            pltpu.sync_copy(acc.at[sl], ob.at[sl])
            pltpu.sync_copy(dn.at[sl], db.at[sl])
        plsc.subcore_barrier()


@functools.lru_cache(maxsize=1)
def _gagg():
  return functools.partial(
    pl.kernel,
    out_type=[jax.ShapeDtypeStruct((NPAD, F), _f32) for _ in range(4)]
             + [jax.ShapeDtypeStruct((NPAD,), _f32) for _ in range(4)],
    mesh=_sc_mesh(),
    compiler_params=pltpu.CompilerParams(needs_layout_passes=False),
    scratch_types=[
        pltpu.VMEM((C,), jnp.int32),
        pltpu.VMEM((C,), jnp.int32),
        pltpu.VMEM((C,), _f32),
        pltpu.VMEM((C, F), _f32),
        pltpu.VMEM((C,), jnp.int32),
        pltpu.VMEM((C,), jnp.int32),
        pltpu.VMEM((C,), _f32),
        pltpu.VMEM((C, F), _f32),
        pltpu.VMEM_SHARED((NPAD, F), _f32),
        pltpu.VMEM_SHARED((NPAD,), _f32),
        pltpu.SemaphoreType.DMA,
        pltpu.SemaphoreType.DMA,
    ],
)(_gagg_body)


def _spmmt_body(r1_h, c1_h, v1_h, r2_h, c2_h, v2_h, r3_h, c3_h, v3_h,
                x1_h, x2_h, x3_h, zeros_h,
                opa, opb,
                ridx, cidx, vv, rows, ridx1, cidx1, vv1, rows1,
                acc, sem, sem1):
    wid, cid, sid = _wid_cid_sid()
    _zero_acc2d(zeros_h, acc, sid)
    plsc.subcore_barrier()
    coo = [(r1_h, c1_h, v1_h), (r2_h, c2_h, v2_h), (r3_h, c3_h, v3_h)]
    for k, xk_h in enumerate([x1_h, x2_h, x3_h]):
        r_h, c_h, v_h = coo[k]

        def chunkpair(g, _):
            b0 = wid * EPT + (2 * g) * C
            b1 = b0 + C
            pltpu.sync_copy(r_h.at[pl.ds(b0, C)], ridx)
            pltpu.sync_copy(c_h.at[pl.ds(b0, C)], cidx)
            pltpu.sync_copy(v_h.at[pl.ds(b0, C)], vv)
            cp0 = pltpu.async_copy(xk_h.at[ridx], rows, sem)
            pltpu.sync_copy(r_h.at[pl.ds(b1, C)], ridx1)
            pltpu.sync_copy(c_h.at[pl.ds(b1, C)], cidx1)
            pltpu.sync_copy(v_h.at[pl.ds(b1, C)], vv1)
            cp1 = pltpu.async_copy(xk_h.at[ridx1], rows1, sem1)
            cp0.wait()
            _scale_rows(rows, vv, C)
            pltpu.sync_copy(rows, acc.at[cidx], add=True)
            cp1.wait()
            _scale_rows(rows1, vv1, C)
            pltpu.sync_copy(rows1, acc.at[cidx1], add=True)
            return 0

        lax.fori_loop(0, NCH // 2, chunkpair, 0)
        base = wid * EPT + (NCH - 1) * C
        pltpu.sync_copy(r_h.at[pl.ds(base, C)], ridx)
        pltpu.sync_copy(c_h.at[pl.ds(base, C)], cidx)
        pltpu.sync_copy(v_h.at[pl.ds(base, C)], vv)
        pltpu.async_copy(xk_h.at[ridx], rows, sem).wait()
        _scale_rows(rows, vv, C)
        pltpu.sync_copy(rows, acc.at[cidx], add=True)
    plsc.subcore_barrier()
    sl = pl.ds(sid * RPT, RPT)
    @pl.when(cid == 0)
    def _():
        pltpu.sync_copy(acc.at[sl], opa.at[sl])
    @pl.when(cid == 1)
    def _():
        pltpu.sync_copy(acc.at[sl], opb.at[sl])


@functools.lru_cache(maxsize=1)
def _spmmt():
  return functools.partial(
    pl.kernel,
    out_type=[jax.ShapeDtypeStruct((NPAD, F), _f32) for _ in range(2)],
    mesh=_sc_mesh(),
    compiler_params=pltpu.CompilerParams(needs_layout_passes=False),
    scratch_types=[
        pltpu.VMEM((C,), jnp.int32),
        pltpu.VMEM((C,), jnp.int32),
        pltpu.VMEM((C,), _f32),
        pltpu.VMEM((C, F), _f32),
        pltpu.VMEM((C,), jnp.int32),
        pltpu.VMEM((C,), jnp.int32),
        pltpu.VMEM((C,), _f32),
        pltpu.VMEM((C, F), _f32),
        pltpu.VMEM_SHARED((NPAD, F), _f32),
        pltpu.SemaphoreType.DMA,
        pltpu.SemaphoreType.DMA,
    ],
)(_spmmt_body)


_BLK = 1024
_GRID = NPAD // _BLK


def _mm_body(x_r, w_r, o_r):
    o_r[...] = jnp.dot(x_r[...], w_r[...], preferred_element_type=_f32)


def _tc_matmul(xp, w):
    return pl.pallas_call(
        _mm_body,
        grid=(_GRID,),
        in_specs=[pl.BlockSpec((_BLK, F), lambda i: (i, 0)),
                  pl.BlockSpec((F, F), lambda i: (0, 0))],
        out_specs=pl.BlockSpec((_BLK, F), lambda i: (i, 0)),
        out_shape=jax.ShapeDtypeStruct((NPAD, F), _f32),
    )(xp, w)


def _mid_body(y1a_r, y1b_r, y2a_r, y2b_r, y3a_r, y3b_r, filt_r, w2_r, w3_r,
              x1_r, h2_r, h3_r):
    x1_r[...] = filt_r[...] * (y1a_r[...] + y1b_r[...])
    h2_r[...] = jnp.dot(y2a_r[...] + y2b_r[...], w2_r[...],
                        preferred_element_type=_f32)
    h3_r[...] = jnp.dot(y3a_r[...] + y3b_r[...], w3_r[...],
                        preferred_element_type=_f32)


def _tc_mid(y1a, y1b, y2a, y2b, y3a, y3b, filt, w2, w3):
    bs = pl.BlockSpec((_BLK, F), lambda i: (i, 0))
    ws = pl.BlockSpec((F, F), lambda i: (0, 0))
    return pl.pallas_call(
        _mid_body,
        grid=(_GRID,),
        in_specs=[bs, bs, bs, bs, bs, bs,
                  pl.BlockSpec((_BLK, 1), lambda i: (i, 0)), ws, ws],
        out_specs=[bs, bs, bs],
        out_shape=[jax.ShapeDtypeStruct((NPAD, F), _f32) for _ in range(3)],
    )(y1a, y1b, y2a, y2b, y3a, y3b, filt, w2, w3)


def _tab_body(h2_r, h3_r, as2_r, ad2_r, as3_r, ad3_r,
              ss2_r, sd2_r, ss3_r, sd3_r, m_r):
    h2 = h2_r[...]
    h3 = h3_r[...]
    ss2 = jnp.sum(h2 * as2_r[...][None, :], axis=1)
    sd2 = jnp.sum(h2 * ad2_r[...][None, :], axis=1)
    ss3 = jnp.sum(h3 * as3_r[...][None, :], axis=1)
    sd3 = jnp.sum(h3 * ad3_r[...][None, :], axis=1)
    ss2_r[...] = ss2
    sd2_r[...] = sd2
    ss3_r[...] = ss3
    sd3_r[...] = sd3
    m2 = jnp.max(ss2) + jnp.max(sd2)
    m3 = jnp.max(ss3) + jnp.max(sd3)
    lane = lax.broadcasted_iota(jnp.int32, (F,), 0)
    m_r[...] = jnp.where(lane == 0, m2, jnp.where(lane == 1, m3, 0.0))


def _tc_tables(h2, h3, as2, ad2, as3, ad3):
    full = pl.BlockSpec((NPAD, F), lambda: (0, 0))
    vec = pl.BlockSpec((F,), lambda: (0,))
    tab = pl.BlockSpec((NPAD,), lambda: (0,))
    return pl.pallas_call(
        _tab_body,
        in_specs=[full, full, vec, vec, vec, vec],
        out_specs=[tab, tab, tab, tab, pl.BlockSpec((F,), lambda: (0,))],
        out_shape=[jax.ShapeDtypeStruct((NPAD,), _f32) for _ in range(4)]
                  + [jax.ShapeDtypeStruct((F,), _f32)],
    )(h2, h3, as2, ad2, as3, ad3)


def _exp_body(a2_r, a3_r, m_r, e2_r, e3_r):
    m = m_r[...]
    e2_r[...] = jnp.exp(a2_r[...] - m[0])
    e3_r[...] = jnp.exp(a3_r[...] - m[1])


def _tc_exp(a2, a3, m):
    full = pl.BlockSpec((EROWS, 128), lambda: (0, 0))
    return pl.pallas_call(
        _exp_body,
        in_specs=[full, full, pl.BlockSpec((F,), lambda: (0,))],
        out_specs=[full, full],
        out_shape=[jax.ShapeDtypeStruct((EROWS, 128), _f32) for _ in range(2)],
    )(a2, a3, m)


def _norm_body(ya_r, yb_r, da_r, db_r, bias_r, o_r):
    dn = da_r[...] + db_r[...] + 1e-16
    v = (ya_r[...] + yb_r[...]) / dn[:, None] + bias_r[...][None, :]
    o_r[...] = jnp.where(v > 0, v, jnp.exp(jnp.minimum(v, 0.0)) - 1.0)


def _tc_norm_elu(ya, yb, da, db, bias):
    bs = pl.BlockSpec((_BLK, F), lambda i: (i, 0))
    ds1 = pl.BlockSpec((_BLK,), lambda i: (i,))
    return pl.pallas_call(
        _norm_body,
        grid=(_GRID,),
        in_specs=[bs, bs, ds1, ds1, pl.BlockSpec((F,), lambda i: (0,))],
        out_specs=bs,
        out_shape=jax.ShapeDtypeStruct((NPAD, F), _f32),
    )(ya, yb, da, db, bias)


def _fin_body(a_r, b_r, bias_r, o_r):
    o_r[...] = a_r[...] + b_r[...] + bias_r[...][None, :]


def _tc_final(a, b, bias):
    bs = pl.BlockSpec((1000, F), lambda i: (i, 0))
    return pl.pallas_call(
        _fin_body,
        grid=(10,),
        in_specs=[bs, bs, pl.BlockSpec((F,), lambda i: (0,))],
        out_specs=bs,
        out_shape=jax.ShapeDtypeStruct((N, F), _f32),
    )(a, b, bias)


def kernel(x, edge_index, d_indices, d_values, weight, filter1, bias,
           W2, a_src2, a_dst2, b2, W3, a_src3, a_dst3, b3):
    n = x.shape[0]
    zeros = jnp.zeros((128, F), _f32)
    zvec = jnp.zeros((128,), _f32)
    dr = [d_indices[k, 0] for k in range(3)]
    dc = [d_indices[k, 1] for k in range(3)]
    dv = [d_values[k] for k in range(3)]
    coo = [a for k in range(3) for a in (dr[k], dc[k], dv[k])]
    xp = jnp.zeros((NPAD, F), _f32).at[:n].set(x)
    filt = jnp.zeros((NPAD, 1), _f32).at[:n].set(filter1)
    loop = jnp.arange(n, dtype=jnp.int32)
    padlen = EGP - EG
    srcg = jnp.concatenate([edge_index[0], loop,
                            jnp.zeros((padlen,), jnp.int32)])
    dstg = jnp.concatenate([edge_index[1], loop,
                            jnp.full((padlen,), n, jnp.int32)])

    xw = _tc_matmul(xp, weight)
    y1a, y1b, y2a, y2b, y3a, y3b = _spmm3()(*coo, xw, zeros)
    x1, h2, h3 = _tc_mid(y1a, y1b, y2a, y2b, y3a, y3b, filt, W2, W3)
    ss2, sd2, ss3, sd3, m = _tc_tables(h2, h3, a_src2, a_dst2, a_src3, a_dst3)
    a2, a3 = _alpha()(srcg, dstg, ss2, sd2, ss3, sd3)
    e2m, e3m = _tc_exp(a2.reshape(EROWS, 128), a3.reshape(EROWS, 128), m)
    e2 = e2m.reshape(EGP)
    e3 = e3m.reshape(EGP)
    (yg2a, yg2b, yg3a, yg3b,
     dn2a, dn2b, dn3a, dn3b) = _gagg()(srcg, dstg, e2, e3, h2, h3,
                                       zeros, zvec)
    x2f = _tc_norm_elu(yg2a, yg2b, dn2a, dn2b, b2)
    x3f = _tc_norm_elu(yg3a, yg3b, dn3a, dn3b, b3)
    opa, opb = _spmmt()(*coo, x1, x2f, x3f, zeros)
    return _tc_final(opa, opb, bias)
